# Initial kernel scaffold; baseline (speedup 1.0000x reference)
import jax
import jax.numpy as jnp
from jax.experimental import pallas as pl


def kernel(x, table):
    raise NotImplementedError("write your pallas kernel here")


# speedup vs baseline: 1.1134x; 1.1134x over previous
import functools

import jax
import jax.numpy as jnp
from jax import lax
from jax.experimental import pallas as pl
from jax.experimental.pallas import tpu as pltpu
from jax.experimental.pallas import tpu_sc as plsc

BATCH = 16384
HIST = 50
EMBED_DIM = 32

N = BATCH * HIST
NUM_WORKERS = 32
PER_WORKER = N // NUM_WORKERS
CHUNK = 1280
NCHUNK = PER_WORKER // CHUNK
NROWBUF = 3
NIDXBUF = 2


def _gather_body(x_hbm, table_hbm, out_hbm,
                 idx0, idx1, rows0, rows1, rows2,
                 gsem0, gsem1, wsem0, wsem1, wsem2):
    wid = lax.axis_index("s") * 2 + lax.axis_index("c")
    base = wid * PER_WORKER

    idx_bufs = (idx0, idx1)
    row_bufs = (rows0, rows1, rows2)
    gsems = (gsem0, gsem1)
    wsems = (wsem0, wsem1, wsem2)

    def load_idx(i):
        pltpu.sync_copy(x_hbm.at[pl.ds(base + i * CHUNK, CHUNK)],
                        idx_bufs[i % NIDXBUF])

    def start_gather(i):
        return pltpu.async_copy(table_hbm.at[idx_bufs[i % NIDXBUF]],
                                row_bufs[i % NROWBUF], gsems[i % NIDXBUF])

    def start_write(i):
        return pltpu.async_copy(row_bufs[i % NROWBUF],
                                out_hbm.at[pl.ds(base + i * CHUNK, CHUNK)],
                                wsems[i % NROWBUF])

    load_idx(0)
    gathers = {0: start_gather(0)}
    writes = {}
    for i in range(NCHUNK):
        if i + 1 < NCHUNK:
            load_idx(i + 1)
            j = i + 1 - NROWBUF
            if j >= 0:
                writes.pop(j).wait()
            gathers[i + 1] = start_gather(i + 1)
        gathers.pop(i).wait()
        writes[i] = start_write(i)
    for j in sorted(writes):
        writes.pop(j).wait()


@jax.jit
def _gather(xf, table):
    mesh = plsc.VectorSubcoreMesh(core_axis_name="c", subcore_axis_name="s")
    return pl.kernel(
        _gather_body,
        out_type=jax.ShapeDtypeStruct((N, EMBED_DIM), jnp.float32),
        mesh=mesh,
        compiler_params=pltpu.CompilerParams(use_tc_tiling_on_sc=False),
        scratch_types=[
            pltpu.VMEM((CHUNK,), jnp.int32),
            pltpu.VMEM((CHUNK,), jnp.int32),
            pltpu.VMEM((CHUNK, EMBED_DIM), jnp.float32),
            pltpu.VMEM((CHUNK, EMBED_DIM), jnp.float32),
            pltpu.VMEM((CHUNK, EMBED_DIM), jnp.float32),
            pltpu.SemaphoreType.DMA,
            pltpu.SemaphoreType.DMA,
            pltpu.SemaphoreType.DMA,
            pltpu.SemaphoreType.DMA,
            pltpu.SemaphoreType.DMA,
        ],
    )(xf, table)


def kernel(x, table):
    xf = x.reshape(-1).astype(jnp.int32)
    out = _gather(xf, table)
    return out.reshape(BATCH, HIST, EMBED_DIM)
